# baseline (speedup 1.0000x reference)
import functools

import jax
import jax.numpy as jnp
from jax import lax
from jax.experimental import pallas as pl
from jax.experimental.pallas import tpu as pltpu
from jax.experimental.pallas import tpu_sc as plsc

VOCAB = 100000
EMB = 128
HID = 128
CTX = 200

NC = 2
NS = 16
NW = NC * NS
BPW = 8
ACTIVE = CTX // BPW
LANES = 16

TILE = 3584
NCORES = 2
G2 = 14
VPAD = NCORES * G2 * TILE


def _sc_gather_sum(idx, table):
    mesh = plsc.VectorSubcoreMesh(core_axis_name="c", subcore_axis_name="s")

    @functools.partial(
        pl.kernel,
        out_type=jax.ShapeDtypeStruct((NW, EMB), jnp.float32),
        mesh=mesh,
        scratch_types=[
            pltpu.VMEM((BPW,), jnp.int32),
            pltpu.VMEM((BPW, EMB), jnp.float32),
            pltpu.VMEM((EMB,), jnp.float32),
            pltpu.SemaphoreType.DMA,
        ],
    )
    def gather_kernel(idx_hbm, table_hbm, out_hbm, idx_v, rows_v, acc_v, sem):
        wid = lax.axis_index("s") * NC + lax.axis_index("c")

        @pl.when(wid < ACTIVE)
        def _work():
            pltpu.sync_copy(idx_hbm.at[pl.ds(wid * BPW, BPW)], idx_v)
            pltpu.async_copy(table_hbm.at[idx_v], rows_v, sem).wait()
            for c in range(EMB // LANES):
                acc = jnp.zeros((LANES,), jnp.float32)
                for j in range(BPW):
                    acc = acc + rows_v[j, pl.ds(c * LANES, LANES)]
                acc_v[pl.ds(c * LANES, LANES)] = acc
            pltpu.sync_copy(acc_v, out_hbm.at[wid])

    return gather_kernel(idx, table)


def _tc_body(part_ref, wp_ref, bp_ref, wo_ref, bo_ref,
             out_ref, stats_ref, h_ref, m_ref, s_ref):
    c = pl.program_id(0)
    g = pl.program_id(1)

    @pl.when(g == 0)
    def _init():
        e = jnp.sum(part_ref[0:ACTIVE, :], axis=0, keepdims=True)
        h = jnp.dot(e, wp_ref[...], preferred_element_type=jnp.float32)
        h_ref[...] = jnp.maximum(h + bp_ref[...], 0.0)
        m_ref[0] = -jnp.inf
        s_ref[0] = 0.0

    logits = jnp.dot(h_ref[...], wo_ref[...],
                     preferred_element_type=jnp.float32) + bo_ref[...]
    cols = (c * G2 + g) * TILE + lax.broadcasted_iota(jnp.int32, (1, TILE), 1)
    lm = jnp.where(cols < VOCAB, logits, -jnp.inf)
    m_old = m_ref[0]
    m_new = jnp.maximum(m_old, jnp.max(lm))
    s_ref[0] = s_ref[0] * jnp.exp(m_old - m_new) + jnp.sum(jnp.exp(lm - m_new))
    m_ref[0] = m_new
    out_ref[...] = logits

    @pl.when(g == G2 - 1)
    def _publish():
        stats_ref[0, 0, 0] = m_ref[0]
        stats_ref[0, 0, 1] = s_ref[0]


def _tc_logits(partials, W_proj, b_proj2d, W_out, b_out2d):
    return pl.pallas_call(
        _tc_body,
        grid=(NCORES, G2),
        in_specs=[
            pl.BlockSpec((NW, EMB), lambda c, g: (0, 0)),
            pl.BlockSpec((EMB, HID), lambda c, g: (0, 0)),
            pl.BlockSpec((1, HID), lambda c, g: (0, 0)),
            pl.BlockSpec((HID, TILE), lambda c, g: (0, c * G2 + g)),
            pl.BlockSpec((1, TILE), lambda c, g: (0, c * G2 + g)),
        ],
        out_specs=[
            pl.BlockSpec((1, TILE), lambda c, g: (0, c * G2 + g)),
            pl.BlockSpec((1, 1, 2), lambda c, g: (c, 0, 0),
                         memory_space=pltpu.SMEM),
        ],
        out_shape=[
            jax.ShapeDtypeStruct((1, VPAD), jnp.float32),
            jax.ShapeDtypeStruct((NCORES, 1, 2), jnp.float32),
        ],
        scratch_shapes=[
            pltpu.VMEM((1, HID), jnp.float32),
            pltpu.SMEM((1,), jnp.float32),
            pltpu.SMEM((1,), jnp.float32),
        ],
        compiler_params=pltpu.CompilerParams(
            dimension_semantics=("parallel", "arbitrary")),
    )(partials, W_proj, b_proj2d, W_out, b_out2d)


def _norm_body(lg_ref, stats_ref, out_ref):
    m0 = stats_ref[0, 0, 0]
    s0 = stats_ref[0, 0, 1]
    m1 = stats_ref[1, 0, 0]
    s1 = stats_ref[1, 0, 1]
    mg = jnp.maximum(m0, m1)
    lse = mg + jnp.log(s0 * jnp.exp(m0 - mg) + s1 * jnp.exp(m1 - mg))
    out_ref[...] = lg_ref[...] - lse


def _tc_normalize(logits2d, stats):
    half = VPAD // NCORES
    return pl.pallas_call(
        _norm_body,
        grid=(NCORES,),
        in_specs=[
            pl.BlockSpec((1, half), lambda c: (0, c)),
            pl.BlockSpec(memory_space=pltpu.SMEM),
        ],
        out_specs=pl.BlockSpec((1, half), lambda c: (0, c)),
        out_shape=jax.ShapeDtypeStruct((1, VPAD), jnp.float32),
        compiler_params=pltpu.CompilerParams(
            dimension_semantics=("parallel",)),
    )(logits2d, stats)


def kernel(inputs, table, W_proj, b_proj, W_out, b_out):
    partials = _sc_gather_sum(inputs.astype(jnp.int32), table)
    logits, stats = _tc_logits(partials, W_proj, b_proj.reshape(1, HID),
                               W_out, b_out.reshape(1, VOCAB))
    out = _tc_normalize(logits, stats)
    return out[:, :VOCAB]


# speedup vs baseline: 1.1354x; 1.1354x over previous
import functools

import jax
import jax.numpy as jnp
from jax import lax
from jax.experimental import pallas as pl
from jax.experimental.pallas import tpu as pltpu
from jax.experimental.pallas import tpu_sc as plsc

VOCAB = 100000
EMB = 128
HID = 128
CTX = 200

NC = 2
NS = 16
NW = NC * NS
BPW = 8
ACTIVE = CTX // BPW
LANES = 16

NSTREAM = 4
TILE = 2048
SLAB = NSTREAM * TILE
GRID = (VOCAB + SLAB - 1) // SLAB
VPAD = GRID * SLAB


def _sc_gather_sum(idx, table):
    mesh = plsc.VectorSubcoreMesh(core_axis_name="c", subcore_axis_name="s")

    @functools.partial(
        pl.kernel,
        out_type=jax.ShapeDtypeStruct((NW, EMB), jnp.float32),
        mesh=mesh,
        scratch_types=[
            pltpu.VMEM((BPW,), jnp.int32),
            pltpu.VMEM((BPW, EMB), jnp.float32),
            pltpu.VMEM((EMB,), jnp.float32),
            pltpu.SemaphoreType.DMA,
        ],
    )
    def gather_kernel(idx_hbm, table_hbm, out_hbm, idx_v, rows_v, acc_v, sem):
        wid = lax.axis_index("s") * NC + lax.axis_index("c")

        @pl.when(wid < ACTIVE)
        def _work():
            pltpu.sync_copy(idx_hbm.at[pl.ds(wid * BPW, BPW)], idx_v)
            pltpu.async_copy(table_hbm.at[idx_v], rows_v, sem).wait()
            for c in range(EMB // LANES):
                acc = jnp.zeros((LANES,), jnp.float32)
                for j in range(BPW):
                    acc = acc + rows_v[j, pl.ds(c * LANES, LANES)]
                acc_v[pl.ds(c * LANES, LANES)] = acc
            pltpu.sync_copy(acc_v, out_hbm.at[wid])

    return gather_kernel(idx, table)


def _tc_body(part_ref, wp_ref, bp_ref, wo0_ref, wo1_ref, wo2_ref, wo3_ref,
             bo_ref, out_ref, h_ref, m_ref, s_ref):
    g = pl.program_id(0)

    @pl.when(g == 0)
    def _init():
        e = jnp.sum(part_ref[0:ACTIVE, :], axis=0, keepdims=True)
        h = jnp.dot(e, wp_ref[...], preferred_element_type=jnp.float32)
        h_ref[...] = jnp.maximum(h + bp_ref[...], 0.0)
        m_ref[0] = -jnp.inf
        s_ref[0] = 0.0

    h = h_ref[...]
    parts = [jnp.dot(h, wo_ref[...], preferred_element_type=jnp.float32)
             for wo_ref in (wo0_ref, wo1_ref, wo2_ref, wo3_ref)]
    logits = jnp.concatenate(parts, axis=1) + bo_ref[...]
    cols = g * SLAB + lax.broadcasted_iota(jnp.int32, (1, SLAB), 1)
    lm = jnp.where(cols < VOCAB, logits, -jnp.inf)
    m_old = m_ref[0]
    m_new = jnp.maximum(m_old, jnp.max(lm))
    s_ref[0] = s_ref[0] * jnp.exp(m_old - m_new) + jnp.sum(jnp.exp(lm - m_new))
    m_ref[0] = m_new
    out_ref[0, pl.ds(g * SLAB, SLAB)] = logits[0]

    @pl.when(g == GRID - 1)
    def _finish():
        out_ref[...] = out_ref[...] - (m_ref[0] + jnp.log(s_ref[0]))


def _tc_mlp_logits(partials, W_proj, b_proj2d, W_out, b_out2d):
    ntiles = (VOCAB + TILE - 1) // TILE
    wo_specs = [
        pl.BlockSpec(
            (HID, TILE),
            functools.partial(
                lambda k, g: (0, jnp.minimum(NSTREAM * g + k, ntiles - 1)), k))
        for k in range(NSTREAM)
    ]
    return pl.pallas_call(
        _tc_body,
        grid=(GRID,),
        in_specs=[
            pl.BlockSpec((NW, EMB), lambda g: (0, 0)),
            pl.BlockSpec((EMB, HID), lambda g: (0, 0)),
            pl.BlockSpec((1, HID), lambda g: (0, 0)),
            *wo_specs,
            pl.BlockSpec((1, SLAB), lambda g: (0, g)),
        ],
        out_specs=pl.BlockSpec((1, VPAD), lambda g: (0, 0)),
        out_shape=jax.ShapeDtypeStruct((1, VPAD), jnp.float32),
        scratch_shapes=[
            pltpu.VMEM((1, HID), jnp.float32),
            pltpu.SMEM((1,), jnp.float32),
            pltpu.SMEM((1,), jnp.float32),
        ],
    )(partials, W_proj, b_proj2d, W_out, W_out, W_out, W_out, b_out2d)


def kernel(inputs, table, W_proj, b_proj, W_out, b_out):
    partials = _sc_gather_sum(inputs.astype(jnp.int32), table)
    out = _tc_mlp_logits(partials, W_proj, b_proj.reshape(1, HID),
                         W_out, b_out.reshape(1, VOCAB))
    return out[:, :VOCAB]


# speedup vs baseline: 1.1802x; 1.0395x over previous
import functools

import jax
import jax.numpy as jnp
from jax import lax
from jax.experimental import pallas as pl
from jax.experimental.pallas import tpu as pltpu
from jax.experimental.pallas import tpu_sc as plsc

VOCAB = 100000
EMB = 128
HID = 128
CTX = 200

NC = 2
NS = 16
NW = NC * NS
BPW = 8
ACTIVE = CTX // BPW
LANES = 16

NSTREAM = 4
TILE = 4096
SLAB = NSTREAM * TILE
GRID = (VOCAB + SLAB - 1) // SLAB
VPAD = GRID * SLAB


def _sc_gather_sum(idx, table):
    mesh = plsc.VectorSubcoreMesh(core_axis_name="c", subcore_axis_name="s")

    @functools.partial(
        pl.kernel,
        out_type=jax.ShapeDtypeStruct((NW, EMB), jnp.float32),
        mesh=mesh,
        scratch_types=[
            pltpu.VMEM((BPW,), jnp.int32),
            pltpu.VMEM((BPW, EMB), jnp.float32),
            pltpu.VMEM((EMB,), jnp.float32),
            pltpu.SemaphoreType.DMA,
        ],
    )
    def gather_kernel(idx_hbm, table_hbm, out_hbm, idx_v, rows_v, acc_v, sem):
        wid = lax.axis_index("s") * NC + lax.axis_index("c")

        @pl.when(wid < ACTIVE)
        def _work():
            pltpu.sync_copy(idx_hbm.at[pl.ds(wid * BPW, BPW)], idx_v)
            pltpu.async_copy(table_hbm.at[idx_v], rows_v, sem).wait()
            for c in range(EMB // LANES):
                acc = jnp.zeros((LANES,), jnp.float32)
                for j in range(BPW):
                    acc = acc + rows_v[j, pl.ds(c * LANES, LANES)]
                acc_v[pl.ds(c * LANES, LANES)] = acc
            pltpu.sync_copy(acc_v, out_hbm.at[wid])

    return gather_kernel(idx, table)


def _tc_body(part_ref, wp_ref, bp_ref, wo0_ref, wo1_ref, wo2_ref, wo3_ref,
             bo_ref, out_ref, h_ref, m_ref, s_ref):
    g = pl.program_id(0)

    @pl.when(g == 0)
    def _init():
        e = jnp.sum(part_ref[0:ACTIVE, :], axis=0, keepdims=True)
        h = jnp.dot(e, wp_ref[...], preferred_element_type=jnp.float32)
        h_ref[...] = jnp.maximum(h + bp_ref[...], 0.0)
        m_ref[0] = -jnp.inf
        s_ref[0] = 0.0

    h = h_ref[...]
    parts = [jnp.dot(h, wo_ref[...], preferred_element_type=jnp.float32)
             for wo_ref in (wo0_ref, wo1_ref, wo2_ref, wo3_ref)]
    logits = jnp.concatenate(parts, axis=1) + bo_ref[...]
    cols = g * SLAB + lax.broadcasted_iota(jnp.int32, (1, SLAB), 1)
    lm = jnp.where(cols < VOCAB, logits, -jnp.inf)
    m_old = m_ref[0]
    m_new = jnp.maximum(m_old, jnp.max(lm))
    s_ref[0] = s_ref[0] * jnp.exp(m_old - m_new) + jnp.sum(jnp.exp(lm - m_new))
    m_ref[0] = m_new
    out_ref[0, pl.ds(g * SLAB, SLAB)] = logits[0]

    @pl.when(g == GRID - 1)
    def _finish():
        out_ref[...] = out_ref[...] - (m_ref[0] + jnp.log(s_ref[0]))


def _tc_mlp_logits(partials, W_proj, b_proj2d, W_out, b_out2d):
    ntiles = (VOCAB + TILE - 1) // TILE
    wo_specs = [
        pl.BlockSpec(
            (HID, TILE),
            functools.partial(
                lambda k, g: (0, jnp.minimum(NSTREAM * g + k, ntiles - 1)), k))
        for k in range(NSTREAM)
    ]
    return pl.pallas_call(
        _tc_body,
        grid=(GRID,),
        in_specs=[
            pl.BlockSpec((NW, EMB), lambda g: (0, 0)),
            pl.BlockSpec((EMB, HID), lambda g: (0, 0)),
            pl.BlockSpec((1, HID), lambda g: (0, 0)),
            *wo_specs,
            pl.BlockSpec((1, SLAB), lambda g: (0, g)),
        ],
        out_specs=pl.BlockSpec((1, VPAD), lambda g: (0, 0)),
        out_shape=jax.ShapeDtypeStruct((1, VPAD), jnp.float32),
        scratch_shapes=[
            pltpu.VMEM((1, HID), jnp.float32),
            pltpu.SMEM((1,), jnp.float32),
            pltpu.SMEM((1,), jnp.float32),
        ],
    )(partials, W_proj, b_proj2d, W_out, W_out, W_out, W_out, b_out2d)


def kernel(inputs, table, W_proj, b_proj, W_out, b_out):
    partials = _sc_gather_sum(inputs.astype(jnp.int32), table)
    out = _tc_mlp_logits(partials, W_proj, b_proj.reshape(1, HID),
                         W_out, b_out.reshape(1, VOCAB))
    return out[:, :VOCAB]


# speedup vs baseline: 1.1817x; 1.0013x over previous
import functools

import jax
import jax.numpy as jnp
from jax import lax
from jax.experimental import pallas as pl
from jax.experimental.pallas import tpu as pltpu
from jax.experimental.pallas import tpu_sc as plsc

VOCAB = 100000
EMB = 128
HID = 128
CTX = 200

NC = 2
NS = 16
NW = NC * NS
BPW = 8
ACTIVE = CTX // BPW
LANES = 16

NSTREAM = 2
TILE = 8192
SLAB = NSTREAM * TILE
GRID = (VOCAB + SLAB - 1) // SLAB
VPAD = GRID * SLAB


def _sc_gather_sum(idx, table):
    mesh = plsc.VectorSubcoreMesh(core_axis_name="c", subcore_axis_name="s")

    @functools.partial(
        pl.kernel,
        out_type=jax.ShapeDtypeStruct((NW, EMB), jnp.float32),
        mesh=mesh,
        scratch_types=[
            pltpu.VMEM((BPW,), jnp.int32),
            pltpu.VMEM((BPW, EMB), jnp.float32),
            pltpu.VMEM((EMB,), jnp.float32),
            pltpu.SemaphoreType.DMA,
        ],
    )
    def gather_kernel(idx_hbm, table_hbm, out_hbm, idx_v, rows_v, acc_v, sem):
        wid = lax.axis_index("s") * NC + lax.axis_index("c")

        @pl.when(wid < ACTIVE)
        def _work():
            pltpu.sync_copy(idx_hbm.at[pl.ds(wid * BPW, BPW)], idx_v)
            pltpu.async_copy(table_hbm.at[idx_v], rows_v, sem).wait()
            for c in range(EMB // LANES):
                acc = jnp.zeros((LANES,), jnp.float32)
                for j in range(BPW):
                    acc = acc + rows_v[j, pl.ds(c * LANES, LANES)]
                acc_v[pl.ds(c * LANES, LANES)] = acc
            pltpu.sync_copy(acc_v, out_hbm.at[wid])

    return gather_kernel(idx, table)


def _tc_body(part_ref, wp_ref, bp_ref, *rest):
    wo_refs = rest[:NSTREAM]
    bo_ref, out_ref, h_ref, m_ref, s_ref = rest[NSTREAM:]
    g = pl.program_id(0)

    @pl.when(g == 0)
    def _init():
        e = jnp.sum(part_ref[0:ACTIVE, :], axis=0, keepdims=True)
        h = jnp.dot(e, wp_ref[...], preferred_element_type=jnp.float32)
        h_ref[...] = jnp.maximum(h + bp_ref[...], 0.0)
        m_ref[0] = -jnp.inf
        s_ref[0] = 0.0

    h = h_ref[...]
    parts = [jnp.dot(h, wo_ref[...], preferred_element_type=jnp.float32)
             for wo_ref in wo_refs]
    logits = jnp.concatenate(parts, axis=1) + bo_ref[...]
    cols = g * SLAB + lax.broadcasted_iota(jnp.int32, (1, SLAB), 1)
    lm = jnp.where(cols < VOCAB, logits, -jnp.inf)
    m_old = m_ref[0]
    m_new = jnp.maximum(m_old, jnp.max(lm))
    s_ref[0] = s_ref[0] * jnp.exp(m_old - m_new) + jnp.sum(jnp.exp(lm - m_new))
    m_ref[0] = m_new
    out_ref[0, pl.ds(g * SLAB, SLAB)] = logits[0]

    @pl.when(g == GRID - 1)
    def _finish():
        out_ref[...] = out_ref[...] - (m_ref[0] + jnp.log(s_ref[0]))


def _tc_mlp_logits(partials, W_proj, b_proj2d, W_out, b_out2d):
    ntiles = (VOCAB + TILE - 1) // TILE
    wo_specs = [
        pl.BlockSpec(
            (HID, TILE),
            functools.partial(
                lambda k, g: (0, jnp.minimum(NSTREAM * g + k, ntiles - 1)), k))
        for k in range(NSTREAM)
    ]
    return pl.pallas_call(
        _tc_body,
        grid=(GRID,),
        in_specs=[
            pl.BlockSpec((NW, EMB), lambda g: (0, 0)),
            pl.BlockSpec((EMB, HID), lambda g: (0, 0)),
            pl.BlockSpec((1, HID), lambda g: (0, 0)),
            *wo_specs,
            pl.BlockSpec((1, SLAB), lambda g: (0, g)),
        ],
        out_specs=pl.BlockSpec((1, VPAD), lambda g: (0, 0)),
        out_shape=jax.ShapeDtypeStruct((1, VPAD), jnp.float32),
        scratch_shapes=[
            pltpu.VMEM((1, HID), jnp.float32),
            pltpu.SMEM((1,), jnp.float32),
            pltpu.SMEM((1,), jnp.float32),
        ],
    )(partials, W_proj, b_proj2d, *([W_out] * NSTREAM), b_out2d)


def kernel(inputs, table, W_proj, b_proj, W_out, b_out):
    partials = _sc_gather_sum(inputs.astype(jnp.int32), table)
    out = _tc_mlp_logits(partials, W_proj, b_proj.reshape(1, HID),
                         W_out, b_out.reshape(1, VOCAB))
    return out[:, :VOCAB]
